# baseline (speedup 1.0000x reference)
import functools
import jax
import jax.numpy as jnp
from jax import lax
from jax.experimental import pallas as pl
from jax.experimental.pallas import tpu as pltpu
from jax.experimental.pallas import tpu_sc as plsc

N = 100000
K = 8
D = 2
E = 3200000
S = 3000

NB = 2048
NGRID = 49
NP = NB * NGRID
SP = 3072
SROWS = 24

NC = 2
NS = 16
L = 16
NW = NC * NS
EPW = E // NW
CHUNK = 1000
NCH = EPW // CHUNK
CAP = 512
CROWS = CAP // 128

f32 = jnp.float32
i32 = jnp.int32

_sc_params = pltpu.CompilerParams(use_tc_tiling_on_sc=False,
                                  needs_layout_passes=False)


def _k1_body(z_ref, gt_ref, u_ref, vl_ref, vs_ref):
    @pl.when(pl.program_id(0) == 0)
    def _():
        u_ref[...] = jnp.zeros_like(u_ref)
        vl_ref[...] = jnp.zeros_like(vl_ref)
        vs_ref[...] = jnp.zeros_like(vs_ref)

    z = z_ref[...]
    zmax = jnp.max(z, axis=0, keepdims=True)
    ez = jnp.exp(z - zmax)
    zs = ez / jnp.sum(ez, axis=0, keepdims=True)
    gs = 1.0 / (1.0 + jnp.exp(-gt_ref[...]))
    zg = zs * gs
    dn = (((1,), (1,)), ((), ()))
    u_ref[...] += lax.dot_general(zs, zg, dn, preferred_element_type=f32)
    vl_ref[...] += lax.dot_general(jnp.ones_like(zs), zg, dn,
                                   preferred_element_type=f32)
    vs_ref[...] += jnp.broadcast_to(jnp.sum(zg, axis=1, keepdims=True), (8, 8))


_k1 = pl.pallas_call(
    _k1_body,
    grid=(NGRID,),
    in_specs=[pl.BlockSpec((8, NB), lambda i: (0, i)),
              pl.BlockSpec((8, NB), lambda i: (0, i))],
    out_specs=[pl.BlockSpec((8, 8), lambda i: (0, 0)),
               pl.BlockSpec((8, 8), lambda i: (0, 0)),
               pl.BlockSpec((8, 8), lambda i: (0, 0))],
    out_shape=[jax.ShapeDtypeStruct((8, 8), f32),
               jax.ShapeDtypeStruct((8, 8), f32),
               jax.ShapeDtypeStruct((8, 8), f32)],
)


def _k2_body(z_ref, gt_ref, u_ref, vl_ref, a_ref, zs_ref, zg_ref, p_ref):
    z = z_ref[...]
    zmax = jnp.max(z, axis=0, keepdims=True)
    ez = jnp.exp(z - zmax)
    zs = ez / jnp.sum(ez, axis=0, keepdims=True)
    gs = 1.0 / (1.0 + jnp.exp(-gt_ref[...]))
    zg = zs * gs
    azc = jnp.dot(a_ref[...], u_ref[...] / vl_ref[...],
                  preferred_element_type=f32)
    p_ref[...] = jnp.dot(azc, zs, preferred_element_type=f32)
    zs_ref[...] = zs
    zg_ref[...] = zg


_k2 = pl.pallas_call(
    _k2_body,
    grid=(NGRID,),
    in_specs=[pl.BlockSpec((8, NB), lambda i: (0, i)),
              pl.BlockSpec((8, NB), lambda i: (0, i)),
              pl.BlockSpec((8, 8), lambda i: (0, 0)),
              pl.BlockSpec((8, 8), lambda i: (0, 0)),
              pl.BlockSpec((8, 8), lambda i: (0, 0))],
    out_specs=[pl.BlockSpec((8, NB), lambda i: (0, i)),
               pl.BlockSpec((8, NB), lambda i: (0, i)),
               pl.BlockSpec((8, NB), lambda i: (0, i))],
    out_shape=[jax.ShapeDtypeStruct((8, NP), f32),
               jax.ShapeDtypeStruct((8, NP), f32),
               jax.ShapeDtypeStruct((8, NP), f32)],
)


SPW = SP // NW


def _sca_body(w_hbm, t_hbm, idx_hbm, ws_hbm, ts_hbm, idx_v, r16, r4, sem):
    wid = lax.axis_index("s") * NC + lax.axis_index("c")
    base = wid * SPW
    pltpu.sync_copy(idx_hbm.at[pl.ds(base, SPW)], idx_v)
    pltpu.async_copy(w_hbm.at[idx_v], r16, sem).wait()
    pltpu.sync_copy(r16, ws_hbm.at[pl.ds(base, SPW)])
    pltpu.async_copy(t_hbm.at[idx_v], r4, sem).wait()
    pltpu.sync_copy(r4, ts_hbm.at[pl.ds(base, SPW)])


def _scc_body(ei_hbm, ej_hbm, sidx_hbm, tab_hbm, out_hbm,
              flags, sidx_v, ib0, jb0, ib1, jb1, si, sj, ri, rj, accb,
              s0, s1):
    wid = lax.axis_index("s") * NC + lax.axis_index("c")
    zf = jnp.zeros((L,), f32)
    zi = jnp.zeros((L,), i32)
    ones = jnp.ones((L,), f32)

    @pl.loop(0, N, step=L)
    def _(o):
        flags[pl.ds(o, L)] = zf

    pltpu.sync_copy(sidx_hbm, sidx_v)

    @pl.loop(0, SP, step=L)
    def _(o):
        plsc.store_scatter(flags, [sidx_v[pl.ds(o, L)]], ones)

    for g in range(CROWS):
        for o in range(0, 128, L):
            si[g, pl.ds(o, L)] = zi
            sj[g, pl.ds(o, L)] = zi

    def issue(c, ibuf, jbuf, sem):
        base = wid * EPW + c * CHUNK
        pltpu.async_copy(ei_hbm.at[pl.ds(base, CHUNK)], ibuf, sem)
        pltpu.async_copy(ej_hbm.at[pl.ds(base, CHUNK)], jbuf, sem)

    def drain(ibuf, jbuf, sem):
        pltpu.make_async_copy(ei_hbm.at[pl.ds(0, CHUNK)], ibuf, sem).wait()
        pltpu.make_async_copy(ej_hbm.at[pl.ds(0, CHUNK)], jbuf, sem).wait()

    def process(ibuf, jbuf, cnt0):
        def vec_body(v, cnt):
            iv = ibuf[pl.ds(v * L, L)]
            jv = jbuf[pl.ds(v * L, L)]
            fi = plsc.load_gather(flags, [iv])
            fj = plsc.load_gather(flags, [jv])
            m = (fi * fj) > 0.5
            npop = plsc.all_reduce_population_count(m)

            def slow(cc):
                mi = m.astype(i32)
                pos = jnp.minimum(cc + jnp.cumsum(mi) - 1, CAP - 1)
                prow = jnp.right_shift(pos, 7)
                pcol = jnp.bitwise_and(pos, 127)
                plsc.store_scatter(si, [prow, pcol], iv, mask=m)
                plsc.store_scatter(sj, [prow, pcol], jv, mask=m)
                return cc + npop[0]

            return lax.cond(npop[0] > 0, slow, lambda cc: cc, cnt)

        return lax.fori_loop(0, CHUNK // L, vec_body, cnt0)

    issue(0, ib0, jb0, s0)

    def outer(c2, cnt):
        c = c2 * 2
        drain(ib0, jb0, s0)
        issue(c + 1, ib1, jb1, s1)
        cnt = process(ib0, jb0, cnt)
        drain(ib1, jb1, s1)

        @pl.when(c + 2 < NCH)
        def _():
            issue(c + 2, ib0, jb0, s0)

        return process(ib1, jb1, cnt)

    cnt = lax.fori_loop(0, NCH // 2, outer, jnp.int32(0))

    for g in range(CROWS):
        @pl.when(cnt > g * 128)
        def _():
            pltpu.async_copy(tab_hbm.at[si.at[g]], ri.at[g], s0)
            pltpu.async_copy(tab_hbm.at[sj.at[g]], rj.at[g], s1)
            pltpu.make_async_copy(tab_hbm.at[si.at[g]], ri.at[g], s0).wait()
            pltpu.make_async_copy(tab_hbm.at[sj.at[g]], rj.at[g], s1).wait()

    iota = lax.iota(i32, L)
    c0 = zi
    c1 = zi + 1
    c2 = zi + 2

    def sgroup(q, acc):
        g = q >> 3
        ro = (q & 7) * L
        gv = jnp.broadcast_to(g, (L,))
        rv = ro + iota
        bi = plsc.load_gather(ri, [gv, rv, c0])
        xi = plsc.load_gather(ri, [gv, rv, c1])
        yi = plsc.load_gather(ri, [gv, rv, c2])
        bj = plsc.load_gather(rj, [gv, rv, c0])
        xj = plsc.load_gather(rj, [gv, rv, c1])
        yj = plsc.load_gather(rj, [gv, rv, c2])
        dxx = xi - xj + 1e-6
        dyy = yi - yj + 1e-6
        x = jnp.maximum(dxx * dxx + dyy * dyy, 1e-30)
        bits = plsc.bitcast(x, i32)
        r = plsc.bitcast(0x5F3759DF - jnp.right_shift(bits, 1), f32)
        hx = 0.5 * x
        r = r * (1.5 - hx * r * r)
        r = r * (1.5 - hx * r * r)
        r = r * (1.5 - hx * r * r)
        dist = x * r
        valid = (q * L + iota) < cnt
        return acc + jnp.where(valid, bi + bj - dist, 0.0)

    ngroups = jnp.right_shift(cnt + (L - 1), 4)
    acc = lax.fori_loop(0, ngroups, sgroup, jnp.zeros((L,), f32))
    accb[...] = acc
    pltpu.sync_copy(accb, out_hbm.at[wid])


@functools.lru_cache(maxsize=1)
def _sc_kernels():
    mesh = plsc.VectorSubcoreMesh(core_axis_name="c", subcore_axis_name="s")
    sca = pl.kernel(
        _sca_body,
        mesh=mesh,
        out_type=[jax.ShapeDtypeStruct((SP, 16), f32),
                  jax.ShapeDtypeStruct((SP, 16), f32)],
        scratch_types=[pltpu.VMEM((SPW,), i32),
                       pltpu.VMEM((SPW, 16), f32),
                       pltpu.VMEM((SPW, 16), f32),
                       pltpu.SemaphoreType.DMA],
        compiler_params=_sc_params,
    )
    scc = pl.kernel(
        _scc_body,
        mesh=mesh,
        out_type=jax.ShapeDtypeStruct((NW, 16), f32),
        scratch_types=[pltpu.VMEM((N,), f32),
                       pltpu.VMEM((SP,), i32),
                       pltpu.VMEM((CHUNK,), i32),
                       pltpu.VMEM((CHUNK,), i32),
                       pltpu.VMEM((CHUNK,), i32),
                       pltpu.VMEM((CHUNK,), i32),
                       pltpu.VMEM((CROWS, 128), i32),
                       pltpu.VMEM((CROWS, 128), i32),
                       pltpu.VMEM((CROWS, 128, 16), f32),
                       pltpu.VMEM((CROWS, 128, 16), f32),
                       pltpu.VMEM((16,), f32),
                       pltpu.SemaphoreType.DMA,
                       pltpu.SemaphoreType.DMA],
        compiler_params=_sc_params,
    )
    return sca, scc


def _k3a_body(wst_ref, vs_ref, a_ref, p_ref):
    lane = lax.broadcasted_iota(i32, (1, SP), 1)
    validc = lane < S
    zs_s = jnp.where(validc, wst_ref[0:8, :], 0.0)
    cs = jnp.where(validc, wst_ref[8:16, :], 0.0) / vs_ref[:, 0:1]
    dn = (((1,), (1,)), ((), ()))
    ks = lax.dot_general(zs_s, cs, dn, preferred_element_type=f32)
    t1 = jnp.dot(ks, zs_s, preferred_element_type=f32)
    p_ref[...] = jnp.dot(a_ref[...], t1, preferred_element_type=f32)


_k3a = pl.pallas_call(
    _k3a_body,
    in_specs=[pl.BlockSpec((16, SP), lambda: (0, 0)),
              pl.BlockSpec((8, 8), lambda: (0, 0)),
              pl.BlockSpec((8, 8), lambda: (0, 0))],
    out_specs=pl.BlockSpec((8, SP), lambda: (0, 0)),
    out_shape=jax.ShapeDtypeStruct((8, SP), f32),
)


def _k3b_body(pxs_ref, pys_ref, bss_ref, pxl_ref, pyl_ref, bsl_ref,
              out_ref):
    rb = pl.program_id(0)

    @pl.when(rb == 0)
    def _():
        out_ref[...] = jnp.zeros((1, 1), f32)

    pxi = pxs_ref[...].reshape(128, 1)
    pyi = pys_ref[...].reshape(128, 1)
    bsi = bss_ref[...].reshape(128, 1)
    dxx = pxi - pxl_ref[...] + 1e-6
    dyy = pyi - pyl_ref[...] + 1e-6
    dist = jnp.sqrt(dxx * dxx + dyy * dyy)
    mat = jnp.exp(bsi + bsl_ref[...] - dist)
    rix = rb * 128 + lax.broadcasted_iota(i32, (128, SP), 0)
    cix = lax.broadcasted_iota(i32, (128, SP), 1)
    keep = (rix != cix) & (rix < S) & (cix < S)
    tile = jnp.sum(jnp.where(keep, mat, 0.0))
    e1 = jnp.exp(f32(1.0))
    out_ref[...] -= (0.5 * e1 * e1 * tile).reshape(1, 1)


_k3b = pl.pallas_call(
    _k3b_body,
    grid=(SROWS,),
    in_specs=[pl.BlockSpec((1, 128, 1), lambda i: (i, 0, 0)),
              pl.BlockSpec((1, 128, 1), lambda i: (i, 0, 0)),
              pl.BlockSpec((1, 128, 1), lambda i: (i, 0, 0)),
              pl.BlockSpec((1, SP), lambda i: (0, 0)),
              pl.BlockSpec((1, SP), lambda i: (0, 0)),
              pl.BlockSpec((1, SP), lambda i: (0, 0))],
    out_specs=pl.BlockSpec((1, 1), lambda i: (0, 0)),
    out_shape=jax.ShapeDtypeStruct((1, 1), f32),
)


def kernel(sampling_weights, edge_index, beta, A, Z, G):
    skey = jax.random.key(42)
    p = sampling_weights / sampling_weights.sum()
    g = jax.random.gumbel(skey, (N,), dtype=f32) + jnp.log(p)
    _, sample_idx = lax.top_k(g, S)
    sidx_pad = jnp.concatenate(
        [sample_idx, jnp.broadcast_to(sample_idx[:1], (SP - S,))]
    ).astype(i32)

    Zp = jnp.pad(Z, ((0, 0), (0, NP - N)))
    GTp = jnp.pad(G.T, ((0, 0), (0, NP - N)), constant_values=-1e30)
    A8 = jnp.concatenate([A, jnp.zeros((8 - D, K), f32)], axis=0)

    U, Vlane, Vsub = _k1(Zp, GTp)
    Zs8, ZG8, P8 = _k2(Zp, GTp, U, Vlane, A8)

    mx = P8[0, :N]
    my = P8[1, :N]
    table = jnp.pad(jnp.stack([beta, mx, my], axis=1), ((0, 0), (0, 13)))
    W = jnp.concatenate([Zs8, ZG8], axis=0)[:, :N].T

    _sca, _scc = _sc_kernels()
    Ws, Ts = _sca(W, table, sidx_pad)
    partial = _scc(edge_index[0], edge_index[1], sidx_pad, table)

    azcz = _k3a(Ws.T, Vsub, A8)
    px = azcz[0]
    py = azcz[1]
    bs = Ts[:, 0]
    neg_z1 = _k3b(px.reshape(SROWS, 128, 1), py.reshape(SROWS, 128, 1),
                  bs.reshape(SROWS, 128, 1), px.reshape(1, SP),
                  py.reshape(1, SP), bs.reshape(1, SP))
    return neg_z1 + jnp.sum(partial).reshape(1, 1)


# speedup vs baseline: 1.0531x; 1.0531x over previous
import functools
import jax
import jax.numpy as jnp
from jax import lax
from jax.experimental import pallas as pl
from jax.experimental.pallas import tpu as pltpu
from jax.experimental.pallas import tpu_sc as plsc

N = 100000
K = 8
D = 2
E = 3200000
S = 3000

NB = 2048
NGRID = 49
NP = NB * NGRID
SP = 3072
SROWS = 24

NC = 2
NS = 16
L = 16
NW = NC * NS
EPW = E // NW
CHUNK = 2000
NCH = EPW // CHUNK
CAP = 512
CROWS = CAP // 128

f32 = jnp.float32
i32 = jnp.int32

_sc_params = pltpu.CompilerParams(use_tc_tiling_on_sc=False,
                                  needs_layout_passes=False)


def _k1_body(z_ref, gt_ref, u_ref, vl_ref, vs_ref):
    @pl.when(pl.program_id(0) == 0)
    def _():
        u_ref[...] = jnp.zeros_like(u_ref)
        vl_ref[...] = jnp.zeros_like(vl_ref)
        vs_ref[...] = jnp.zeros_like(vs_ref)

    z = z_ref[...]
    zmax = jnp.max(z, axis=0, keepdims=True)
    ez = jnp.exp(z - zmax)
    zs = ez / jnp.sum(ez, axis=0, keepdims=True)
    gs = 1.0 / (1.0 + jnp.exp(-gt_ref[...]))
    zg = zs * gs
    dn = (((1,), (1,)), ((), ()))
    u_ref[...] += lax.dot_general(zs, zg, dn, preferred_element_type=f32)
    vl_ref[...] += lax.dot_general(jnp.ones_like(zs), zg, dn,
                                   preferred_element_type=f32)
    vs_ref[...] += jnp.broadcast_to(jnp.sum(zg, axis=1, keepdims=True), (8, 8))


_k1 = pl.pallas_call(
    _k1_body,
    grid=(NGRID,),
    in_specs=[pl.BlockSpec((8, NB), lambda i: (0, i)),
              pl.BlockSpec((8, NB), lambda i: (0, i))],
    out_specs=[pl.BlockSpec((8, 8), lambda i: (0, 0)),
               pl.BlockSpec((8, 8), lambda i: (0, 0)),
               pl.BlockSpec((8, 8), lambda i: (0, 0))],
    out_shape=[jax.ShapeDtypeStruct((8, 8), f32),
               jax.ShapeDtypeStruct((8, 8), f32),
               jax.ShapeDtypeStruct((8, 8), f32)],
)


def _k2_body(z_ref, gt_ref, u_ref, vl_ref, a_ref, zs_ref, zg_ref, p_ref):
    z = z_ref[...]
    zmax = jnp.max(z, axis=0, keepdims=True)
    ez = jnp.exp(z - zmax)
    zs = ez / jnp.sum(ez, axis=0, keepdims=True)
    gs = 1.0 / (1.0 + jnp.exp(-gt_ref[...]))
    zg = zs * gs
    azc = jnp.dot(a_ref[...], u_ref[...] / vl_ref[...],
                  preferred_element_type=f32)
    p_ref[...] = jnp.dot(azc, zs, preferred_element_type=f32)
    zs_ref[...] = zs
    zg_ref[...] = zg


_k2 = pl.pallas_call(
    _k2_body,
    grid=(NGRID,),
    in_specs=[pl.BlockSpec((8, NB), lambda i: (0, i)),
              pl.BlockSpec((8, NB), lambda i: (0, i)),
              pl.BlockSpec((8, 8), lambda i: (0, 0)),
              pl.BlockSpec((8, 8), lambda i: (0, 0)),
              pl.BlockSpec((8, 8), lambda i: (0, 0))],
    out_specs=[pl.BlockSpec((8, NB), lambda i: (0, i)),
               pl.BlockSpec((8, NB), lambda i: (0, i)),
               pl.BlockSpec((8, NB), lambda i: (0, i))],
    out_shape=[jax.ShapeDtypeStruct((8, NP), f32),
               jax.ShapeDtypeStruct((8, NP), f32),
               jax.ShapeDtypeStruct((8, NP), f32)],
)


SPW = SP // NW


def _sca_body(w_hbm, t_hbm, idx_hbm, ws_hbm, ts_hbm, idx_v, r16, r4, sem):
    wid = lax.axis_index("s") * NC + lax.axis_index("c")
    base = wid * SPW
    pltpu.sync_copy(idx_hbm.at[pl.ds(base, SPW)], idx_v)
    pltpu.async_copy(w_hbm.at[idx_v], r16, sem).wait()
    pltpu.sync_copy(r16, ws_hbm.at[pl.ds(base, SPW)])
    pltpu.async_copy(t_hbm.at[idx_v], r4, sem).wait()
    pltpu.sync_copy(r4, ts_hbm.at[pl.ds(base, SPW)])


def _scc_body(e_hbm, sidx_hbm, tab_hbm, out_hbm,
              flags, sidx_v, ib0, jb0, ib1, jb1, si, sj, ri, rj, accb,
              s0, s1):
    wid = lax.axis_index("s") * NC + lax.axis_index("c")
    zf = jnp.zeros((L,), f32)
    zi = jnp.zeros((L,), i32)
    ones = jnp.ones((L,), f32)

    @pl.loop(0, N, step=L)
    def _(o):
        flags[pl.ds(o, L)] = zf

    pltpu.sync_copy(sidx_hbm, sidx_v)

    @pl.loop(0, SP, step=L)
    def _(o):
        plsc.store_scatter(flags, [sidx_v[pl.ds(o, L)]], ones)

    for g in range(CROWS):
        for o in range(0, 128, L):
            si[g, pl.ds(o, L)] = zi
            sj[g, pl.ds(o, L)] = zi

    def issue(c, ibuf, jbuf, sem):
        base = wid * EPW + c * CHUNK
        pltpu.async_copy(e_hbm.at[0, pl.ds(base, CHUNK)], ibuf, sem)
        pltpu.async_copy(e_hbm.at[1, pl.ds(base, CHUNK)], jbuf, sem)

    def drain(ibuf, jbuf, sem):
        pltpu.make_async_copy(e_hbm.at[0, pl.ds(0, CHUNK)], ibuf, sem).wait()
        pltpu.make_async_copy(e_hbm.at[1, pl.ds(0, CHUNK)], jbuf, sem).wait()

    def process(ibuf, jbuf, cnt0):
        def vec_body(v, cnt):
            iv = ibuf[pl.ds(v * L, L)]
            jv = jbuf[pl.ds(v * L, L)]
            fi = plsc.load_gather(flags, [iv])
            fj = plsc.load_gather(flags, [jv])
            m = (fi * fj) > 0.5
            npop = plsc.all_reduce_population_count(m)

            def slow(cc):
                mi = m.astype(i32)
                pos = jnp.minimum(cc + jnp.cumsum(mi) - 1, CAP - 1)
                prow = jnp.right_shift(pos, 7)
                pcol = jnp.bitwise_and(pos, 127)
                plsc.store_scatter(si, [prow, pcol], iv, mask=m)
                plsc.store_scatter(sj, [prow, pcol], jv, mask=m)
                return cc + npop[0]

            return lax.cond(npop[0] > 0, slow, lambda cc: cc, cnt)

        def vec5(w, cnt):
            for u in range(5):
                cnt = vec_body(w * 5 + u, cnt)
            return cnt

        return lax.fori_loop(0, CHUNK // L // 5, vec5, cnt0)

    issue(0, ib0, jb0, s0)

    def outer(c2, cnt):
        c = c2 * 2
        drain(ib0, jb0, s0)
        issue(c + 1, ib1, jb1, s1)
        cnt = process(ib0, jb0, cnt)
        drain(ib1, jb1, s1)

        @pl.when(c + 2 < NCH)
        def _():
            issue(c + 2, ib0, jb0, s0)

        return process(ib1, jb1, cnt)

    cnt = lax.fori_loop(0, NCH // 2, outer, jnp.int32(0))

    for g in range(CROWS):
        @pl.when(cnt > g * 128)
        def _():
            pltpu.async_copy(tab_hbm.at[si.at[g]], ri.at[g], s0)
            pltpu.async_copy(tab_hbm.at[sj.at[g]], rj.at[g], s1)
            pltpu.make_async_copy(tab_hbm.at[si.at[g]], ri.at[g], s0).wait()
            pltpu.make_async_copy(tab_hbm.at[sj.at[g]], rj.at[g], s1).wait()

    iota = lax.iota(i32, L)
    c0 = zi
    c1 = zi + 1
    c2 = zi + 2

    def sgroup(q, acc):
        g = q >> 3
        ro = (q & 7) * L
        gv = jnp.broadcast_to(g, (L,))
        rv = ro + iota
        bi = plsc.load_gather(ri, [gv, rv, c0])
        xi = plsc.load_gather(ri, [gv, rv, c1])
        yi = plsc.load_gather(ri, [gv, rv, c2])
        bj = plsc.load_gather(rj, [gv, rv, c0])
        xj = plsc.load_gather(rj, [gv, rv, c1])
        yj = plsc.load_gather(rj, [gv, rv, c2])
        dxx = xi - xj + 1e-6
        dyy = yi - yj + 1e-6
        x = jnp.maximum(dxx * dxx + dyy * dyy, 1e-30)
        bits = plsc.bitcast(x, i32)
        r = plsc.bitcast(0x5F3759DF - jnp.right_shift(bits, 1), f32)
        hx = 0.5 * x
        r = r * (1.5 - hx * r * r)
        r = r * (1.5 - hx * r * r)
        r = r * (1.5 - hx * r * r)
        dist = x * r
        valid = (q * L + iota) < cnt
        return acc + jnp.where(valid, bi + bj - dist, 0.0)

    ngroups = jnp.right_shift(cnt + (L - 1), 4)
    acc = lax.fori_loop(0, ngroups, sgroup, jnp.zeros((L,), f32))
    accb[...] = acc
    pltpu.sync_copy(accb, out_hbm.at[wid])


@functools.lru_cache(maxsize=1)
def _sc_kernels():
    mesh = plsc.VectorSubcoreMesh(core_axis_name="c", subcore_axis_name="s")
    sca = pl.kernel(
        _sca_body,
        mesh=mesh,
        out_type=[jax.ShapeDtypeStruct((SP, 16), f32),
                  jax.ShapeDtypeStruct((SP, 16), f32)],
        scratch_types=[pltpu.VMEM((SPW,), i32),
                       pltpu.VMEM((SPW, 16), f32),
                       pltpu.VMEM((SPW, 16), f32),
                       pltpu.SemaphoreType.DMA],
        compiler_params=_sc_params,
    )
    scc = pl.kernel(
        _scc_body,
        mesh=mesh,
        out_type=jax.ShapeDtypeStruct((NW, 16), f32),
        scratch_types=[pltpu.VMEM((N,), f32),
                       pltpu.VMEM((SP,), i32),
                       pltpu.VMEM((CHUNK,), i32),
                       pltpu.VMEM((CHUNK,), i32),
                       pltpu.VMEM((CHUNK,), i32),
                       pltpu.VMEM((CHUNK,), i32),
                       pltpu.VMEM((CROWS, 128), i32),
                       pltpu.VMEM((CROWS, 128), i32),
                       pltpu.VMEM((CROWS, 128, 16), f32),
                       pltpu.VMEM((CROWS, 128, 16), f32),
                       pltpu.VMEM((16,), f32),
                       pltpu.SemaphoreType.DMA,
                       pltpu.SemaphoreType.DMA],
        compiler_params=_sc_params,
    )
    return sca, scc


def _k3a_body(wst_ref, vs_ref, a_ref, p_ref):
    lane = lax.broadcasted_iota(i32, (1, SP), 1)
    validc = lane < S
    zs_s = jnp.where(validc, wst_ref[0:8, :], 0.0)
    cs = jnp.where(validc, wst_ref[8:16, :], 0.0) / vs_ref[:, 0:1]
    dn = (((1,), (1,)), ((), ()))
    ks = lax.dot_general(zs_s, cs, dn, preferred_element_type=f32)
    t1 = jnp.dot(ks, zs_s, preferred_element_type=f32)
    p_ref[...] = jnp.dot(a_ref[...], t1, preferred_element_type=f32)


_k3a = pl.pallas_call(
    _k3a_body,
    in_specs=[pl.BlockSpec((16, SP), lambda: (0, 0)),
              pl.BlockSpec((8, 8), lambda: (0, 0)),
              pl.BlockSpec((8, 8), lambda: (0, 0))],
    out_specs=pl.BlockSpec((8, SP), lambda: (0, 0)),
    out_shape=jax.ShapeDtypeStruct((8, SP), f32),
)


def _k3b_body(pxs_ref, pys_ref, bss_ref, pxl_ref, pyl_ref, bsl_ref,
              out_ref):
    rb = pl.program_id(0)

    @pl.when(rb == 0)
    def _():
        out_ref[...] = jnp.zeros((1, 1), f32)

    pxi = pxs_ref[...].reshape(128, 1)
    pyi = pys_ref[...].reshape(128, 1)
    bsi = bss_ref[...].reshape(128, 1)
    dxx = pxi - pxl_ref[...] + 1e-6
    dyy = pyi - pyl_ref[...] + 1e-6
    dist = jnp.sqrt(dxx * dxx + dyy * dyy)
    mat = jnp.exp(bsi + bsl_ref[...] - dist)
    rix = rb * 128 + lax.broadcasted_iota(i32, (128, SP), 0)
    cix = lax.broadcasted_iota(i32, (128, SP), 1)
    keep = (rix != cix) & (rix < S) & (cix < S)
    tile = jnp.sum(jnp.where(keep, mat, 0.0))
    e1 = jnp.exp(f32(1.0))
    out_ref[...] -= (0.5 * e1 * e1 * tile).reshape(1, 1)


_k3b = pl.pallas_call(
    _k3b_body,
    grid=(SROWS,),
    in_specs=[pl.BlockSpec((1, 128, 1), lambda i: (i, 0, 0)),
              pl.BlockSpec((1, 128, 1), lambda i: (i, 0, 0)),
              pl.BlockSpec((1, 128, 1), lambda i: (i, 0, 0)),
              pl.BlockSpec((1, SP), lambda i: (0, 0)),
              pl.BlockSpec((1, SP), lambda i: (0, 0)),
              pl.BlockSpec((1, SP), lambda i: (0, 0))],
    out_specs=pl.BlockSpec((1, 1), lambda i: (0, 0)),
    out_shape=jax.ShapeDtypeStruct((1, 1), f32),
)


def kernel(sampling_weights, edge_index, beta, A, Z, G):
    skey = jax.random.key(42)
    p = sampling_weights / sampling_weights.sum()
    g = jax.random.gumbel(skey, (N,), dtype=f32) + jnp.log(p)
    _, sample_idx = lax.top_k(g, S)
    sidx_pad = jnp.concatenate(
        [sample_idx, jnp.broadcast_to(sample_idx[:1], (SP - S,))]
    ).astype(i32)

    Zp = jnp.pad(Z, ((0, 0), (0, NP - N)))
    GTp = jnp.pad(G.T, ((0, 0), (0, NP - N)), constant_values=-1e30)
    A8 = jnp.concatenate([A, jnp.zeros((8 - D, K), f32)], axis=0)

    U, Vlane, Vsub = _k1(Zp, GTp)
    Zs8, ZG8, P8 = _k2(Zp, GTp, U, Vlane, A8)

    mx = P8[0, :N]
    my = P8[1, :N]
    table = jnp.pad(jnp.stack([beta, mx, my], axis=1), ((0, 0), (0, 13)))
    W = jnp.concatenate([Zs8, ZG8], axis=0)[:, :N].T

    _sca, _scc = _sc_kernels()
    Ws, Ts = _sca(W, table, sidx_pad)
    partial = _scc(edge_index, sidx_pad, table)

    azcz = _k3a(Ws.T, Vsub, A8)
    px = azcz[0]
    py = azcz[1]
    bs = Ts[:, 0]
    neg_z1 = _k3b(px.reshape(SROWS, 128, 1), py.reshape(SROWS, 128, 1),
                  bs.reshape(SROWS, 128, 1), px.reshape(1, SP),
                  py.reshape(1, SP), bs.reshape(1, SP))
    return neg_z1 + jnp.sum(partial).reshape(1, 1)


# speedup vs baseline: 1.3754x; 1.3060x over previous
import functools
import jax
import jax.numpy as jnp
from jax import lax
from jax.experimental import pallas as pl
from jax.experimental.pallas import tpu as pltpu
from jax.experimental.pallas import tpu_sc as plsc

N = 100000
K = 8
D = 2
E = 3200000
S = 3000

NB = 2048
NGRID = 49
NP = NB * NGRID
SP = 3072
SROWS = 24

NC = 2
NS = 16
L = 16
NW = NC * NS
EPW = E // NW
CHUNK = 2000
NCH = EPW // CHUNK
CAP = 512
CROWS = CAP // 128

f32 = jnp.float32
i32 = jnp.int32

_sc_params = pltpu.CompilerParams(use_tc_tiling_on_sc=False,
                                  needs_layout_passes=False)


def _k1_body(z_ref, gt_ref, u_ref, vl_ref, vs_ref):
    @pl.when(pl.program_id(0) == 0)
    def _():
        u_ref[...] = jnp.zeros_like(u_ref)
        vl_ref[...] = jnp.zeros_like(vl_ref)
        vs_ref[...] = jnp.zeros_like(vs_ref)

    z = z_ref[...]
    zmax = jnp.max(z, axis=0, keepdims=True)
    ez = jnp.exp(z - zmax)
    zs = ez / jnp.sum(ez, axis=0, keepdims=True)
    gs = 1.0 / (1.0 + jnp.exp(-gt_ref[...]))
    zg = zs * gs
    dn = (((1,), (1,)), ((), ()))
    u_ref[...] += lax.dot_general(zs, zg, dn, preferred_element_type=f32)
    vl_ref[...] += lax.dot_general(jnp.ones_like(zs), zg, dn,
                                   preferred_element_type=f32)
    vs_ref[...] += jnp.broadcast_to(jnp.sum(zg, axis=1, keepdims=True), (8, 8))


_k1 = pl.pallas_call(
    _k1_body,
    grid=(NGRID,),
    in_specs=[pl.BlockSpec((8, NB), lambda i: (0, i)),
              pl.BlockSpec((8, NB), lambda i: (0, i))],
    out_specs=[pl.BlockSpec((8, 8), lambda i: (0, 0)),
               pl.BlockSpec((8, 8), lambda i: (0, 0)),
               pl.BlockSpec((8, 8), lambda i: (0, 0))],
    out_shape=[jax.ShapeDtypeStruct((8, 8), f32),
               jax.ShapeDtypeStruct((8, 8), f32),
               jax.ShapeDtypeStruct((8, 8), f32)],
)


def _k2_body(z_ref, gt_ref, u_ref, vl_ref, a_ref, zs_ref, zg_ref, p_ref):
    z = z_ref[...]
    zmax = jnp.max(z, axis=0, keepdims=True)
    ez = jnp.exp(z - zmax)
    zs = ez / jnp.sum(ez, axis=0, keepdims=True)
    gs = 1.0 / (1.0 + jnp.exp(-gt_ref[...]))
    zg = zs * gs
    azc = jnp.dot(a_ref[...], u_ref[...] / vl_ref[...],
                  preferred_element_type=f32)
    p_ref[...] = jnp.dot(azc, zs, preferred_element_type=f32)
    zs_ref[...] = zs
    zg_ref[...] = zg


_k2 = pl.pallas_call(
    _k2_body,
    grid=(NGRID,),
    in_specs=[pl.BlockSpec((8, NB), lambda i: (0, i)),
              pl.BlockSpec((8, NB), lambda i: (0, i)),
              pl.BlockSpec((8, 8), lambda i: (0, 0)),
              pl.BlockSpec((8, 8), lambda i: (0, 0)),
              pl.BlockSpec((8, 8), lambda i: (0, 0))],
    out_specs=[pl.BlockSpec((8, NB), lambda i: (0, i)),
               pl.BlockSpec((8, NB), lambda i: (0, i)),
               pl.BlockSpec((8, NB), lambda i: (0, i))],
    out_shape=[jax.ShapeDtypeStruct((8, NP), f32),
               jax.ShapeDtypeStruct((8, NP), f32),
               jax.ShapeDtypeStruct((8, NP), f32)],
)


SPW = SP // NW


def _sca_body(w_hbm, t_hbm, idx_hbm, ws_hbm, ts_hbm, idx_v, r16, r4, sem):
    wid = lax.axis_index("s") * NC + lax.axis_index("c")
    base = wid * SPW
    pltpu.sync_copy(idx_hbm.at[pl.ds(base, SPW)], idx_v)
    pltpu.async_copy(w_hbm.at[idx_v], r16, sem).wait()
    pltpu.sync_copy(r16, ws_hbm.at[pl.ds(base, SPW)])
    pltpu.async_copy(t_hbm.at[idx_v], r4, sem).wait()
    pltpu.sync_copy(r4, ts_hbm.at[pl.ds(base, SPW)])


def _scc_body(e_hbm, sidx_hbm, tab_hbm, out_hbm,
              flags, sidx_v, ib0, jb0, ib1, jb1, si, sj, ri, rj, accb,
              s0, s1):
    wid = lax.axis_index("s") * NC + lax.axis_index("c")
    zf = jnp.zeros((L,), f32)
    zi = jnp.zeros((L,), i32)
    ones = jnp.ones((L,), f32)

    @pl.loop(0, N, step=L)
    def _(o):
        flags[pl.ds(o, L)] = zf

    pltpu.sync_copy(sidx_hbm, sidx_v)

    @pl.loop(0, SP, step=L)
    def _(o):
        plsc.store_scatter(flags, [sidx_v[pl.ds(o, L)]], ones)

    for g in range(CROWS):
        for o in range(0, 128, L):
            si[g, pl.ds(o, L)] = zi
            sj[g, pl.ds(o, L)] = zi

    def issue(c, ibuf, jbuf, sem):
        base = wid * EPW + c * CHUNK
        pltpu.async_copy(e_hbm.at[0, pl.ds(base, CHUNK)], ibuf, sem)
        pltpu.async_copy(e_hbm.at[1, pl.ds(base, CHUNK)], jbuf, sem)

    def drain(ibuf, jbuf, sem):
        pltpu.make_async_copy(e_hbm.at[0, pl.ds(0, CHUNK)], ibuf, sem).wait()
        pltpu.make_async_copy(e_hbm.at[1, pl.ds(0, CHUNK)], jbuf, sem).wait()

    def process(ibuf, jbuf, cnt0):
        def vec5(w, cnt):
            vb = w * (5 * L)
            ivs, jvs, ms = [], [], []
            for u in range(5):
                iv = ibuf[pl.ds(vb + u * L, L)]
                jv = jbuf[pl.ds(vb + u * L, L)]
                fi = plsc.load_gather(flags, [iv])
                fj = plsc.load_gather(flags, [jv])
                ivs.append(iv)
                jvs.append(jv)
                ms.append((fi * fj) > 0.5)
            mor = ms[0] | ms[1] | ms[2] | ms[3] | ms[4]
            npop = plsc.all_reduce_population_count(mor)

            def slow(cc):
                for u in range(5):
                    mi = ms[u].astype(i32)
                    pc = plsc.all_reduce_population_count(ms[u])
                    pos = jnp.minimum(cc + jnp.cumsum(mi) - 1, CAP - 1)
                    prow = jnp.right_shift(pos, 7)
                    pcol = jnp.bitwise_and(pos, 127)
                    plsc.store_scatter(si, [prow, pcol], ivs[u], mask=ms[u])
                    plsc.store_scatter(sj, [prow, pcol], jvs[u], mask=ms[u])
                    cc = cc + pc[0]
                return cc

            return lax.cond(npop[0] > 0, slow, lambda cc: cc, cnt)

        return lax.fori_loop(0, CHUNK // L // 5, vec5, cnt0)

    issue(0, ib0, jb0, s0)

    def outer(c2, cnt):
        c = c2 * 2
        drain(ib0, jb0, s0)
        issue(c + 1, ib1, jb1, s1)
        cnt = process(ib0, jb0, cnt)
        drain(ib1, jb1, s1)

        @pl.when(c + 2 < NCH)
        def _():
            issue(c + 2, ib0, jb0, s0)

        return process(ib1, jb1, cnt)

    cnt = lax.fori_loop(0, NCH // 2, outer, jnp.int32(0))

    for g in range(CROWS):
        @pl.when(cnt > g * 128)
        def _():
            pltpu.async_copy(tab_hbm.at[si.at[g]], ri.at[g], s0)
            pltpu.async_copy(tab_hbm.at[sj.at[g]], rj.at[g], s1)
            pltpu.make_async_copy(tab_hbm.at[si.at[g]], ri.at[g], s0).wait()
            pltpu.make_async_copy(tab_hbm.at[sj.at[g]], rj.at[g], s1).wait()

    iota = lax.iota(i32, L)
    c0 = zi
    c1 = zi + 1
    c2 = zi + 2

    def sgroup(q, acc):
        g = q >> 3
        ro = (q & 7) * L
        gv = jnp.broadcast_to(g, (L,))
        rv = ro + iota
        bi = plsc.load_gather(ri, [gv, rv, c0])
        xi = plsc.load_gather(ri, [gv, rv, c1])
        yi = plsc.load_gather(ri, [gv, rv, c2])
        bj = plsc.load_gather(rj, [gv, rv, c0])
        xj = plsc.load_gather(rj, [gv, rv, c1])
        yj = plsc.load_gather(rj, [gv, rv, c2])
        dxx = xi - xj + 1e-6
        dyy = yi - yj + 1e-6
        x = jnp.maximum(dxx * dxx + dyy * dyy, 1e-30)
        bits = plsc.bitcast(x, i32)
        r = plsc.bitcast(0x5F3759DF - jnp.right_shift(bits, 1), f32)
        hx = 0.5 * x
        r = r * (1.5 - hx * r * r)
        r = r * (1.5 - hx * r * r)
        r = r * (1.5 - hx * r * r)
        dist = x * r
        valid = (q * L + iota) < cnt
        return acc + jnp.where(valid, bi + bj - dist, 0.0)

    ngroups = jnp.right_shift(cnt + (L - 1), 4)
    acc = lax.fori_loop(0, ngroups, sgroup, jnp.zeros((L,), f32))
    accb[...] = acc
    pltpu.sync_copy(accb, out_hbm.at[wid])


@functools.lru_cache(maxsize=1)
def _sc_kernels():
    mesh = plsc.VectorSubcoreMesh(core_axis_name="c", subcore_axis_name="s")
    sca = pl.kernel(
        _sca_body,
        mesh=mesh,
        out_type=[jax.ShapeDtypeStruct((SP, 16), f32),
                  jax.ShapeDtypeStruct((SP, 16), f32)],
        scratch_types=[pltpu.VMEM((SPW,), i32),
                       pltpu.VMEM((SPW, 16), f32),
                       pltpu.VMEM((SPW, 16), f32),
                       pltpu.SemaphoreType.DMA],
        compiler_params=_sc_params,
    )
    scc = pl.kernel(
        _scc_body,
        mesh=mesh,
        out_type=jax.ShapeDtypeStruct((NW, 16), f32),
        scratch_types=[pltpu.VMEM((N,), f32),
                       pltpu.VMEM((SP,), i32),
                       pltpu.VMEM((CHUNK,), i32),
                       pltpu.VMEM((CHUNK,), i32),
                       pltpu.VMEM((CHUNK,), i32),
                       pltpu.VMEM((CHUNK,), i32),
                       pltpu.VMEM((CROWS, 128), i32),
                       pltpu.VMEM((CROWS, 128), i32),
                       pltpu.VMEM((CROWS, 128, 16), f32),
                       pltpu.VMEM((CROWS, 128, 16), f32),
                       pltpu.VMEM((16,), f32),
                       pltpu.SemaphoreType.DMA,
                       pltpu.SemaphoreType.DMA],
        compiler_params=_sc_params,
    )
    return sca, scc


def _k3a_body(wst_ref, vs_ref, a_ref, p_ref):
    lane = lax.broadcasted_iota(i32, (1, SP), 1)
    validc = lane < S
    zs_s = jnp.where(validc, wst_ref[0:8, :], 0.0)
    cs = jnp.where(validc, wst_ref[8:16, :], 0.0) / vs_ref[:, 0:1]
    dn = (((1,), (1,)), ((), ()))
    ks = lax.dot_general(zs_s, cs, dn, preferred_element_type=f32)
    t1 = jnp.dot(ks, zs_s, preferred_element_type=f32)
    p_ref[...] = jnp.dot(a_ref[...], t1, preferred_element_type=f32)


_k3a = pl.pallas_call(
    _k3a_body,
    in_specs=[pl.BlockSpec((16, SP), lambda: (0, 0)),
              pl.BlockSpec((8, 8), lambda: (0, 0)),
              pl.BlockSpec((8, 8), lambda: (0, 0))],
    out_specs=pl.BlockSpec((8, SP), lambda: (0, 0)),
    out_shape=jax.ShapeDtypeStruct((8, SP), f32),
)


def _k3b_body(pxs_ref, pys_ref, bss_ref, pxl_ref, pyl_ref, bsl_ref,
              out_ref):
    rb = pl.program_id(0)

    @pl.when(rb == 0)
    def _():
        out_ref[...] = jnp.zeros((1, 1), f32)

    pxi = pxs_ref[...].reshape(128, 1)
    pyi = pys_ref[...].reshape(128, 1)
    bsi = bss_ref[...].reshape(128, 1)
    dxx = pxi - pxl_ref[...] + 1e-6
    dyy = pyi - pyl_ref[...] + 1e-6
    dist = jnp.sqrt(dxx * dxx + dyy * dyy)
    mat = jnp.exp(bsi + bsl_ref[...] - dist)
    rix = rb * 128 + lax.broadcasted_iota(i32, (128, SP), 0)
    cix = lax.broadcasted_iota(i32, (128, SP), 1)
    keep = (rix != cix) & (rix < S) & (cix < S)
    tile = jnp.sum(jnp.where(keep, mat, 0.0))
    e1 = jnp.exp(f32(1.0))
    out_ref[...] -= (0.5 * e1 * e1 * tile).reshape(1, 1)


_k3b = pl.pallas_call(
    _k3b_body,
    grid=(SROWS,),
    in_specs=[pl.BlockSpec((1, 128, 1), lambda i: (i, 0, 0)),
              pl.BlockSpec((1, 128, 1), lambda i: (i, 0, 0)),
              pl.BlockSpec((1, 128, 1), lambda i: (i, 0, 0)),
              pl.BlockSpec((1, SP), lambda i: (0, 0)),
              pl.BlockSpec((1, SP), lambda i: (0, 0)),
              pl.BlockSpec((1, SP), lambda i: (0, 0))],
    out_specs=pl.BlockSpec((1, 1), lambda i: (0, 0)),
    out_shape=jax.ShapeDtypeStruct((1, 1), f32),
)


def kernel(sampling_weights, edge_index, beta, A, Z, G):
    skey = jax.random.key(42)
    p = sampling_weights / sampling_weights.sum()
    g = jax.random.gumbel(skey, (N,), dtype=f32) + jnp.log(p)
    _, sample_idx = lax.top_k(g, S)
    sidx_pad = jnp.concatenate(
        [sample_idx, jnp.broadcast_to(sample_idx[:1], (SP - S,))]
    ).astype(i32)

    Zp = jnp.pad(Z, ((0, 0), (0, NP - N)))
    GTp = jnp.pad(G.T, ((0, 0), (0, NP - N)), constant_values=-1e30)
    A8 = jnp.concatenate([A, jnp.zeros((8 - D, K), f32)], axis=0)

    U, Vlane, Vsub = _k1(Zp, GTp)
    Zs8, ZG8, P8 = _k2(Zp, GTp, U, Vlane, A8)

    mx = P8[0, :N]
    my = P8[1, :N]
    table = jnp.pad(jnp.stack([beta, mx, my], axis=1), ((0, 0), (0, 13)))
    W = jnp.concatenate([Zs8, ZG8], axis=0)[:, :N].T

    _sca, _scc = _sc_kernels()
    Ws, Ts = _sca(W, table, sidx_pad)
    partial = _scc(edge_index, sidx_pad, table)

    azcz = _k3a(Ws.T, Vsub, A8)
    px = azcz[0]
    py = azcz[1]
    bs = Ts[:, 0]
    neg_z1 = _k3b(px.reshape(SROWS, 128, 1), py.reshape(SROWS, 128, 1),
                  bs.reshape(SROWS, 128, 1), px.reshape(1, SP),
                  py.reshape(1, SP), bs.reshape(1, SP))
    return neg_z1 + jnp.sum(partial).reshape(1, 1)


# speedup vs baseline: 1.5242x; 1.1082x over previous
import functools
import jax
import jax.numpy as jnp
from jax import lax
from jax.experimental import pallas as pl
from jax.experimental.pallas import tpu as pltpu
from jax.experimental.pallas import tpu_sc as plsc

N = 100000
K = 8
D = 2
E = 3200000
S = 3000

NB = 2048
NGRID = 49
NP = NB * NGRID
SP = 3072
SROWS = 24

NC = 2
NS = 16
L = 16
NW = NC * NS
EPW = E // NW
CHUNK = 2000
NCH = EPW // CHUNK
CAP = 512
CROWS = CAP // 128

f32 = jnp.float32
i32 = jnp.int32

_sc_params = pltpu.CompilerParams(use_tc_tiling_on_sc=False,
                                  needs_layout_passes=False)


def _k1_body(z_ref, gt_ref, u_ref, vl_ref, vs_ref):
    @pl.when(pl.program_id(0) == 0)
    def _():
        u_ref[...] = jnp.zeros_like(u_ref)
        vl_ref[...] = jnp.zeros_like(vl_ref)
        vs_ref[...] = jnp.zeros_like(vs_ref)

    z = z_ref[...]
    zmax = jnp.max(z, axis=0, keepdims=True)
    ez = jnp.exp(z - zmax)
    zs = ez / jnp.sum(ez, axis=0, keepdims=True)
    gs = 1.0 / (1.0 + jnp.exp(-gt_ref[...]))
    zg = zs * gs
    dn = (((1,), (1,)), ((), ()))
    u_ref[...] += lax.dot_general(zs, zg, dn, preferred_element_type=f32)
    vl_ref[...] += lax.dot_general(jnp.ones_like(zs), zg, dn,
                                   preferred_element_type=f32)
    vs_ref[...] += jnp.broadcast_to(jnp.sum(zg, axis=1, keepdims=True), (8, 8))


_k1 = pl.pallas_call(
    _k1_body,
    grid=(NGRID,),
    in_specs=[pl.BlockSpec((8, NB), lambda i: (0, i)),
              pl.BlockSpec((8, NB), lambda i: (0, i))],
    out_specs=[pl.BlockSpec((8, 8), lambda i: (0, 0)),
               pl.BlockSpec((8, 8), lambda i: (0, 0)),
               pl.BlockSpec((8, 8), lambda i: (0, 0))],
    out_shape=[jax.ShapeDtypeStruct((8, 8), f32),
               jax.ShapeDtypeStruct((8, 8), f32),
               jax.ShapeDtypeStruct((8, 8), f32)],
)


def _k2_body(z_ref, gt_ref, u_ref, vl_ref, a_ref, beta_ref, w_ref, tbl_ref):
    z = z_ref[...]
    zmax = jnp.max(z, axis=0, keepdims=True)
    ez = jnp.exp(z - zmax)
    zs = ez / jnp.sum(ez, axis=0, keepdims=True)
    gs = 1.0 / (1.0 + jnp.exp(-gt_ref[...]))
    zg = zs * gs
    azc = jnp.dot(a_ref[...], u_ref[...] / vl_ref[...],
                  preferred_element_type=f32)
    p = jnp.dot(azc, zs, preferred_element_type=f32)
    w_ref[...] = jnp.concatenate([zs, zg], axis=0).T
    t3 = jnp.concatenate([beta_ref[...].reshape(1, NB), p[0:1, :],
                          p[1:2, :]], axis=0).T
    tbl_ref[...] = jnp.concatenate([t3, jnp.zeros((NB, 13), f32)], axis=1)


_k2 = pl.pallas_call(
    _k2_body,
    grid=(NGRID,),
    in_specs=[pl.BlockSpec((8, NB), lambda i: (0, i)),
              pl.BlockSpec((8, NB), lambda i: (0, i)),
              pl.BlockSpec((8, 8), lambda i: (0, 0)),
              pl.BlockSpec((8, 8), lambda i: (0, 0)),
              pl.BlockSpec((8, 8), lambda i: (0, 0)),
              pl.BlockSpec((1, 1, NB), lambda i: (i, 0, 0))],
    out_specs=[pl.BlockSpec((NB, 16), lambda i: (i, 0)),
               pl.BlockSpec((NB, 16), lambda i: (i, 0))],
    out_shape=[jax.ShapeDtypeStruct((NP, 16), f32),
               jax.ShapeDtypeStruct((NP, 16), f32)],
)


SPW = SP // NW


def _sca_body(w_hbm, t_hbm, idx_hbm, ws_hbm, ts_hbm, idx_v, r16, r4, sem):
    wid = lax.axis_index("s") * NC + lax.axis_index("c")
    base = wid * SPW
    pltpu.sync_copy(idx_hbm.at[pl.ds(base, SPW)], idx_v)
    pltpu.async_copy(w_hbm.at[idx_v], r16, sem).wait()
    pltpu.sync_copy(r16, ws_hbm.at[pl.ds(base, SPW)])
    pltpu.async_copy(t_hbm.at[idx_v], r4, sem).wait()
    pltpu.sync_copy(r4, ts_hbm.at[pl.ds(base, SPW)])


def _scc_body(e_hbm, sidx_hbm, tab_hbm, out_hbm,
              flags, sidx_v, ib0, jb0, ib1, jb1, si, sj, ri, rj, accb,
              s0, s1):
    wid = lax.axis_index("s") * NC + lax.axis_index("c")
    zf = jnp.zeros((L,), f32)
    zi = jnp.zeros((L,), i32)
    ones = jnp.ones((L,), f32)

    @pl.loop(0, N, step=L)
    def _(o):
        flags[pl.ds(o, L)] = zf

    pltpu.sync_copy(sidx_hbm, sidx_v)

    @pl.loop(0, SP, step=L)
    def _(o):
        plsc.store_scatter(flags, [sidx_v[pl.ds(o, L)]], ones)

    for g in range(CROWS):
        for o in range(0, 128, L):
            si[g, pl.ds(o, L)] = zi
            sj[g, pl.ds(o, L)] = zi

    def issue(c, ibuf, jbuf, sem):
        base = wid * EPW + c * CHUNK
        pltpu.async_copy(e_hbm.at[0, pl.ds(base, CHUNK)], ibuf, sem)
        pltpu.async_copy(e_hbm.at[1, pl.ds(base, CHUNK)], jbuf, sem)

    def drain(ibuf, jbuf, sem):
        pltpu.make_async_copy(e_hbm.at[0, pl.ds(0, CHUNK)], ibuf, sem).wait()
        pltpu.make_async_copy(e_hbm.at[1, pl.ds(0, CHUNK)], jbuf, sem).wait()

    def process(ibuf, jbuf, cnt0):
        def vec5(w, cnt):
            vb = w * (5 * L)
            ivs, jvs, ms = [], [], []
            for u in range(5):
                iv = ibuf[pl.ds(vb + u * L, L)]
                jv = jbuf[pl.ds(vb + u * L, L)]
                fi = plsc.load_gather(flags, [iv])
                fj = plsc.load_gather(flags, [jv])
                ivs.append(iv)
                jvs.append(jv)
                ms.append((fi * fj) > 0.5)
            mor = ms[0] | ms[1] | ms[2] | ms[3] | ms[4]
            npop = plsc.all_reduce_population_count(mor)

            def slow(cc):
                for u in range(5):
                    mi = ms[u].astype(i32)
                    pc = plsc.all_reduce_population_count(ms[u])
                    pos = jnp.minimum(cc + jnp.cumsum(mi) - 1, CAP - 1)
                    prow = jnp.right_shift(pos, 7)
                    pcol = jnp.bitwise_and(pos, 127)
                    plsc.store_scatter(si, [prow, pcol], ivs[u], mask=ms[u])
                    plsc.store_scatter(sj, [prow, pcol], jvs[u], mask=ms[u])
                    cc = cc + pc[0]
                return cc

            return lax.cond(npop[0] > 0, slow, lambda cc: cc, cnt)

        return lax.fori_loop(0, CHUNK // L // 5, vec5, cnt0)

    issue(0, ib0, jb0, s0)

    def outer(c2, cnt):
        c = c2 * 2
        drain(ib0, jb0, s0)
        issue(c + 1, ib1, jb1, s1)
        cnt = process(ib0, jb0, cnt)
        drain(ib1, jb1, s1)

        @pl.when(c + 2 < NCH)
        def _():
            issue(c + 2, ib0, jb0, s0)

        return process(ib1, jb1, cnt)

    cnt = lax.fori_loop(0, NCH // 2, outer, jnp.int32(0))

    for g in range(CROWS):
        @pl.when(cnt > g * 128)
        def _():
            pltpu.async_copy(tab_hbm.at[si.at[g]], ri.at[g], s0)
            pltpu.async_copy(tab_hbm.at[sj.at[g]], rj.at[g], s1)
            pltpu.make_async_copy(tab_hbm.at[si.at[g]], ri.at[g], s0).wait()
            pltpu.make_async_copy(tab_hbm.at[sj.at[g]], rj.at[g], s1).wait()

    iota = lax.iota(i32, L)
    c0 = zi
    c1 = zi + 1
    c2 = zi + 2

    def sgroup(q, acc):
        g = q >> 3
        ro = (q & 7) * L
        gv = jnp.broadcast_to(g, (L,))
        rv = ro + iota
        bi = plsc.load_gather(ri, [gv, rv, c0])
        xi = plsc.load_gather(ri, [gv, rv, c1])
        yi = plsc.load_gather(ri, [gv, rv, c2])
        bj = plsc.load_gather(rj, [gv, rv, c0])
        xj = plsc.load_gather(rj, [gv, rv, c1])
        yj = plsc.load_gather(rj, [gv, rv, c2])
        dxx = xi - xj + 1e-6
        dyy = yi - yj + 1e-6
        x = jnp.maximum(dxx * dxx + dyy * dyy, 1e-30)
        bits = plsc.bitcast(x, i32)
        r = plsc.bitcast(0x5F3759DF - jnp.right_shift(bits, 1), f32)
        hx = 0.5 * x
        r = r * (1.5 - hx * r * r)
        r = r * (1.5 - hx * r * r)
        r = r * (1.5 - hx * r * r)
        dist = x * r
        valid = (q * L + iota) < cnt
        return acc + jnp.where(valid, bi + bj - dist, 0.0)

    ngroups = jnp.right_shift(cnt + (L - 1), 4)
    acc = lax.fori_loop(0, ngroups, sgroup, jnp.zeros((L,), f32))
    accb[...] = acc
    pltpu.sync_copy(accb, out_hbm.at[wid])


@functools.lru_cache(maxsize=1)
def _sc_kernels():
    mesh = plsc.VectorSubcoreMesh(core_axis_name="c", subcore_axis_name="s")
    sca = pl.kernel(
        _sca_body,
        mesh=mesh,
        out_type=[jax.ShapeDtypeStruct((SP, 16), f32),
                  jax.ShapeDtypeStruct((SP, 16), f32)],
        scratch_types=[pltpu.VMEM((SPW,), i32),
                       pltpu.VMEM((SPW, 16), f32),
                       pltpu.VMEM((SPW, 16), f32),
                       pltpu.SemaphoreType.DMA],
        compiler_params=_sc_params,
    )
    scc = pl.kernel(
        _scc_body,
        mesh=mesh,
        out_type=jax.ShapeDtypeStruct((NW, 16), f32),
        scratch_types=[pltpu.VMEM((N,), f32),
                       pltpu.VMEM((SP,), i32),
                       pltpu.VMEM((CHUNK,), i32),
                       pltpu.VMEM((CHUNK,), i32),
                       pltpu.VMEM((CHUNK,), i32),
                       pltpu.VMEM((CHUNK,), i32),
                       pltpu.VMEM((CROWS, 128), i32),
                       pltpu.VMEM((CROWS, 128), i32),
                       pltpu.VMEM((CROWS, 128, 16), f32),
                       pltpu.VMEM((CROWS, 128, 16), f32),
                       pltpu.VMEM((16,), f32),
                       pltpu.SemaphoreType.DMA,
                       pltpu.SemaphoreType.DMA],
        compiler_params=_sc_params,
    )
    return sca, scc


def _k3a_body(wst_ref, vs_ref, a_ref, p_ref):
    lane = lax.broadcasted_iota(i32, (1, SP), 1)
    validc = lane < S
    zs_s = jnp.where(validc, wst_ref[0:8, :], 0.0)
    cs = jnp.where(validc, wst_ref[8:16, :], 0.0) / vs_ref[:, 0:1]
    dn = (((1,), (1,)), ((), ()))
    ks = lax.dot_general(zs_s, cs, dn, preferred_element_type=f32)
    t1 = jnp.dot(ks, zs_s, preferred_element_type=f32)
    p_ref[...] = jnp.dot(a_ref[...], t1, preferred_element_type=f32)


_k3a = pl.pallas_call(
    _k3a_body,
    in_specs=[pl.BlockSpec((16, SP), lambda: (0, 0)),
              pl.BlockSpec((8, 8), lambda: (0, 0)),
              pl.BlockSpec((8, 8), lambda: (0, 0))],
    out_specs=pl.BlockSpec((8, SP), lambda: (0, 0)),
    out_shape=jax.ShapeDtypeStruct((8, SP), f32),
)


def _k3b_body(pxs_ref, pys_ref, bss_ref, pxl_ref, pyl_ref, bsl_ref,
              out_ref):
    rb = pl.program_id(0)

    @pl.when(rb == 0)
    def _():
        out_ref[...] = jnp.zeros((1, 1), f32)

    pxi = pxs_ref[...].reshape(128, 1)
    pyi = pys_ref[...].reshape(128, 1)
    bsi = bss_ref[...].reshape(128, 1)
    dxx = pxi - pxl_ref[...] + 1e-6
    dyy = pyi - pyl_ref[...] + 1e-6
    dist = jnp.sqrt(dxx * dxx + dyy * dyy)
    mat = jnp.exp(bsi + bsl_ref[...] - dist)
    rix = rb * 128 + lax.broadcasted_iota(i32, (128, SP), 0)
    cix = lax.broadcasted_iota(i32, (128, SP), 1)
    keep = (rix != cix) & (rix < S) & (cix < S)
    tile = jnp.sum(jnp.where(keep, mat, 0.0))
    e1 = jnp.exp(f32(1.0))
    out_ref[...] -= (0.5 * e1 * e1 * tile).reshape(1, 1)


_k3b = pl.pallas_call(
    _k3b_body,
    grid=(SROWS,),
    in_specs=[pl.BlockSpec((1, 128, 1), lambda i: (i, 0, 0)),
              pl.BlockSpec((1, 128, 1), lambda i: (i, 0, 0)),
              pl.BlockSpec((1, 128, 1), lambda i: (i, 0, 0)),
              pl.BlockSpec((1, SP), lambda i: (0, 0)),
              pl.BlockSpec((1, SP), lambda i: (0, 0)),
              pl.BlockSpec((1, SP), lambda i: (0, 0))],
    out_specs=pl.BlockSpec((1, 1), lambda i: (0, 0)),
    out_shape=jax.ShapeDtypeStruct((1, 1), f32),
)


def kernel(sampling_weights, edge_index, beta, A, Z, G):
    skey = jax.random.key(42)
    p = sampling_weights / sampling_weights.sum()
    g = jax.random.gumbel(skey, (N,), dtype=f32) + jnp.log(p)
    _, sample_idx = lax.top_k(g, S)
    sidx_pad = jnp.concatenate(
        [sample_idx, jnp.broadcast_to(sample_idx[:1], (SP - S,))]
    ).astype(i32)

    Zp = jnp.pad(Z, ((0, 0), (0, NP - N)))
    GTp = jnp.pad(G.T, ((0, 0), (0, NP - N)), constant_values=-1e30)
    A8 = jnp.concatenate([A, jnp.zeros((8 - D, K), f32)], axis=0)

    U, Vlane, Vsub = _k1(Zp, GTp)
    beta_p = jnp.pad(beta, (0, NP - N)).reshape(NGRID, 1, NB)
    W, table = _k2(Zp, GTp, U, Vlane, A8, beta_p)

    _sca, _scc = _sc_kernels()
    Ws, Ts = _sca(W, table, sidx_pad)
    partial = _scc(edge_index, sidx_pad, table)

    azcz = _k3a(Ws.T, Vsub, A8)
    px = azcz[0]
    py = azcz[1]
    bs = Ts[:, 0]
    neg_z1 = _k3b(px.reshape(SROWS, 128, 1), py.reshape(SROWS, 128, 1),
                  bs.reshape(SROWS, 128, 1), px.reshape(1, SP),
                  py.reshape(1, SP), bs.reshape(1, SP))
    return neg_z1 + jnp.sum(partial).reshape(1, 1)
